# baseline (speedup 1.0000x reference)
import functools

import jax
import jax.numpy as jnp
from jax import lax
from jax.experimental import pallas as pl
from jax.experimental.pallas import tpu as pltpu
from jax.experimental.pallas import tpu_sc as plsc

EMBED = 64
PADDED = 128
LANES = 16
NUM_CORES = 2
NUM_SUBCORES = 16
NUM_WORKERS = NUM_CORES * NUM_SUBCORES
CHUNK = 128
NBUF = 4
NPACK = 2


def _make_gather(total):
    assert total % (NUM_WORKERS * CHUNK) == 0 and CHUNK % 8 == 0
    per_worker = total // NUM_WORKERS
    num_chunks = per_worker // CHUNK
    assert num_chunks % NBUF == 0 and num_chunks > 2 * NBUF

    mesh = plsc.VectorSubcoreMesh(
        core_axis_name="c", subcore_axis_name="s",
        num_cores=NUM_CORES, num_subcores=NUM_SUBCORES)

    @functools.partial(
        pl.kernel,
        out_type=jax.ShapeDtypeStruct((total, EMBED), jnp.float32),
        mesh=mesh,
        scratch_types=[
            [pltpu.VMEM((CHUNK,), jnp.int32) for _ in range(NBUF)],
            [pltpu.VMEM((CHUNK, PADDED), jnp.float32) for _ in range(NBUF)],
            [pltpu.VMEM((CHUNK, EMBED), jnp.float32) for _ in range(NPACK)],
            [pltpu.SemaphoreType.DMA for _ in range(NBUF)],
            [pltpu.SemaphoreType.DMA for _ in range(NBUF)],
            [pltpu.SemaphoreType.DMA for _ in range(NPACK)],
        ],
    )
    def gather_kernel(idx_hbm, pe_hbm, out_hbm,
                      idxs, rows, packs, isems, gsems, osems):
        wid = lax.axis_index("s") * NUM_CORES + lax.axis_index("c")
        base = wid * per_worker

        def start_idx(g, s):
            pltpu.async_copy(
                idx_hbm.at[pl.ds(base + g * CHUNK, CHUNK)], idxs[s], isems[s])

        def wait_idx(s):
            pltpu.make_async_copy(
                idx_hbm.at[pl.ds(0, CHUNK)], idxs[s], isems[s]).wait()

        def start_gather(s):
            pltpu.async_copy(pe_hbm.at[idxs[s]], rows[s], gsems[s])

        def wait_gather(s):
            pltpu.make_async_copy(pe_hbm.at[idxs[s]], rows[s], gsems[s]).wait()

        def retile(s, p):
            @pl.loop(0, CHUNK, unroll=8)
            def _(r):
                for j in range(EMBED // LANES):
                    packs[p][r, pl.ds(j * LANES, LANES)] = (
                        rows[s][r, pl.ds(j * LANES, LANES)])

        def start_out(g, p):
            pltpu.async_copy(
                packs[p], out_hbm.at[pl.ds(base + g * CHUNK, CHUNK)], osems[p])

        def wait_out(p):
            pltpu.make_async_copy(
                packs[p], out_hbm.at[pl.ds(0, CHUNK)], osems[p]).wait()

        for s in range(NBUF):
            start_idx(s, s)
        for s in range(NBUF - 1):
            wait_idx(s)
            start_gather(s)

        @pl.loop(0, num_chunks, step=NBUF)
        def _(g0):
            for k in range(NBUF):
                s = k
                t = (k + NBUF - 1) % NBUF
                p = k % NPACK
                g = g0 + k

                @pl.when(g + NBUF - 1 < num_chunks)
                def _():
                    wait_idx(t)
                    start_gather(t)

                wait_gather(s)

                @pl.when(g >= NPACK)
                def _():
                    wait_out(p)

                retile(s, p)
                start_out(g, p)

                @pl.when(g + NBUF < num_chunks)
                def _():
                    start_idx(g + NBUF, s)

        for p in range(NPACK):
            wait_out(p)

    return gather_kernel


def kernel(timestep, pe):
    lead_shape = timestep.shape
    idx = timestep.reshape(-1)
    pe_padded = jnp.pad(pe, ((0, 0), (0, PADDED - EMBED)))
    out = _make_gather(idx.shape[0])(idx, pe_padded)
    return out.reshape(*lead_shape, EMBED)
